# Initial kernel scaffold; baseline (speedup 1.0000x reference)
import jax
import jax.numpy as jnp
from jax.experimental import pallas as pl


def kernel(x, edge_index, edge_attr, W1, b1, W2, b2, Wmu, bmu, Wsig, bsig, Wmu2, bmu2, Wsig2, bsig2, edges, high):
    raise NotImplementedError("write your pallas kernel here")


# speedup vs baseline: 9.3175x; 9.3175x over previous
import jax
import jax.numpy as jnp
from jax import lax
from jax.experimental import pallas as pl

_C = 2048
_NSLOT = 64


def _edge_head_body(csrow, vmrow, scol, xs, cxs, cxd, cea,
                    w1a, w1b, w1c, b1r, w2, b2r, wh, bhr, wh2, bh2r, out_ref):
    f32 = jnp.float32
    h = (jnp.dot(cxs[...], w1a[...], preferred_element_type=f32)
         + jnp.dot(cxd[...], w1b[...], preferred_element_type=f32)
         + jnp.dot(cea[...], w1c[...], preferred_element_type=f32)
         + b1r[...])
    u = jnp.maximum(h, 0.0)
    mt = jnp.where(scol[...] == csrow[...], vmrow[...], 0.0)
    acc = jnp.dot(mt, u, preferred_element_type=f32)
    cnt = jnp.dot(mt, jnp.ones((_C, 1), f32), preferred_element_type=f32)
    conv = jnp.dot(acc, w2[...], preferred_element_type=f32) + cnt * b2r[...]
    xsa = xs[...]
    ef = jnp.concatenate([xsa[0:24], conv[0:24], xsa[24:48], conv[24:48]],
                         axis=1)
    z = jnp.dot(ef, wh[...], preferred_element_type=f32) + bhr[...]
    a = jax.nn.softplus(z + 1e-20) + 1e-20
    r1 = a[:, 0:1] / (a[:, 0:1] + a[:, 1:2])
    t = jnp.concatenate([xsa[48:56], conv[48:56]], axis=1)
    z2 = jnp.dot(t, wh2[...], preferred_element_type=f32) + bh2r[...]
    a2 = jax.nn.softplus(z2 + 1e-20) + 1e-20
    r2 = a2[:, 0:1] / (a2[:, 0:1] + a2[:, 1:2])
    out_ref[0:24, :] = jnp.broadcast_to(r1, (24, 128))
    out_ref[24:32, :] = jnp.broadcast_to(r2, (8, 128))


def kernel(x, edge_index, edge_attr, W1, b1, W2, b2, Wmu, bmu, Wsig, bsig,
           Wmu2, bmu2, Wsig2, bsig2, edges, high):
    f32 = jnp.float32
    n_nodes = x.shape[0]
    src = edge_index[0]
    dst = edge_index[1]
    tail_ids = jnp.arange(n_nodes - 3, n_nodes, dtype=jnp.int32)
    slots = jnp.concatenate([edges[:, 0], edges[:, 1], tail_ids])
    slots_pad = jnp.concatenate(
        [slots, jnp.full((_NSLOT - 51,), -1, jnp.int32)])

    mask = jnp.any(src[:, None] == slots[None, :], axis=1)
    n_rel = jnp.sum(mask.astype(jnp.int32))
    idx = jnp.nonzero(mask, size=_C, fill_value=0)[0]
    vm = (jnp.arange(_C) < n_rel).astype(f32)

    cs = src[idx]
    cxs = x[cs]
    cxd = x[dst[idx]]
    cea = edge_attr[idx]
    xs = x[jnp.clip(slots_pad, 0, n_nodes - 1)]

    w1a, w1b, w1c = W1[0:128], W1[128:256], W1[256:272]
    b1r = b1.reshape(1, 32)
    b2r = b2.reshape(1, 32)
    wh = jnp.zeros((320, 128), f32).at[:, 0].set(Wmu[:, 0]).at[:, 1].set(Wsig[:, 0])
    bhr = jnp.zeros((1, 128), f32).at[0, 0].set(bmu[0]).at[0, 1].set(bsig[0])
    wh2 = jnp.zeros((160, 128), f32).at[:, 0].set(Wmu2[:, 0]).at[:, 1].set(Wsig2[:, 0])
    bh2r = jnp.zeros((1, 128), f32).at[0, 0].set(bmu2[0]).at[0, 1].set(bsig2[0])

    csrow = cs.reshape(1, _C)
    vmrow = vm.reshape(1, _C)
    scol = slots_pad.reshape(_NSLOT, 1)

    def _fast(_):
        out = pl.pallas_call(
            _edge_head_body,
            out_shape=jax.ShapeDtypeStruct((32, 128), f32),
        )(csrow, vmrow, scol, xs, cxs, cxd, cea,
          w1a, w1b, w1c, b1r, W2, b2r, wh, bhr, wh2, bh2r)
        r1 = out[0:24, 0]
        r2 = out[24:27, 0]
        return jnp.concatenate([r1 * high[:-3], r2 * high[-3:]])[None, :]

    def _dense(_):
        tmp = jnp.concatenate([x[src], x[dst], edge_attr], axis=1)
        msg = jax.nn.relu(tmp @ W1 + b1) @ W2 + b2
        conv = jax.ops.segment_sum(msg, src, num_segments=n_nodes)
        x_pp = jnp.concatenate([x, conv], axis=1).reshape(-1, n_nodes, 160)
        ef = jnp.concatenate([x_pp[:, edges[:, 0], :], x_pp[:, edges[:, 1], :]],
                             axis=2)
        alpha = jax.nn.softplus(ef @ Wmu + bmu + 1e-20)[..., 0] + 1e-20
        beta = jax.nn.softplus(ef @ Wsig + bsig + 1e-20)[..., 0] + 1e-20
        tail = x_pp[:, -3:, :]
        alpha2 = jax.nn.softplus(tail @ Wmu2 + bmu2 + 1e-20)[..., 0] + 1e-20
        beta2 = jax.nn.softplus(tail @ Wsig2 + bsig2 + 1e-20)[..., 0] + 1e-20
        dis_action = alpha / (alpha + beta) * high[:-3]
        order_act = alpha2 / (alpha2 + beta2) * high[-3:]
        return jnp.concatenate([dis_action, order_act], axis=-1)

    return lax.cond(n_rel <= _C, _fast, _dense, operand=None)


# speedup vs baseline: 10.5639x; 1.1338x over previous
import jax
import jax.numpy as jnp
from jax import lax
from jax.experimental import pallas as pl

_C = 2048
_NSLOT = 64


def _edge_head_body(csrow, cscol, vmrow, vmcol, urow, ucol, scol, xs, xu,
                    cxd, cea, w1a, w1b, w1c, b1r, w2, b2r, wh, bhr, wh2, bh2r,
                    out_ref):
    f32 = jnp.float32
    au = jnp.dot(xu[...], w1a[...], preferred_element_type=f32)
    mc = jnp.where(cscol[...] == urow[...], vmcol[...], 0.0)
    h = (jnp.dot(mc, au, preferred_element_type=f32)
         + jnp.dot(cxd[...], w1b[...], preferred_element_type=f32)
         + jnp.dot(cea[...], w1c[...], preferred_element_type=f32)
         + b1r[...])
    u = jnp.maximum(h, 0.0)
    mu = jnp.where(ucol[...] == csrow[...], vmrow[...], 0.0)
    accu = jnp.dot(mu, u, preferred_element_type=f32)
    cntu = jnp.dot(mu, jnp.ones((_C, 1), f32), preferred_element_type=f32)
    convu = jnp.dot(accu, w2[...], preferred_element_type=f32) + cntu * b2r[...]
    lk = jnp.where(scol[...] == urow[...], 1.0, 0.0)
    conv = jnp.dot(lk, convu, preferred_element_type=f32)
    xsa = xs[...]
    ef = jnp.concatenate([xsa[0:24], conv[0:24], xsa[24:48], conv[24:48]],
                         axis=1)
    z = jnp.dot(ef, wh[...], preferred_element_type=f32) + bhr[...]
    a = jax.nn.softplus(z + 1e-20) + 1e-20
    r1 = a[:, 0:1] / (a[:, 0:1] + a[:, 1:2])
    t = jnp.concatenate([xsa[48:56], conv[48:56]], axis=1)
    z2 = jnp.dot(t, wh2[...], preferred_element_type=f32) + bh2r[...]
    a2 = jax.nn.softplus(z2 + 1e-20) + 1e-20
    r2 = a2[:, 0:1] / (a2[:, 0:1] + a2[:, 1:2])
    out_ref[0:24, :] = jnp.broadcast_to(r1, (24, 128))
    out_ref[24:32, :] = jnp.broadcast_to(r2, (8, 128))


def kernel(x, edge_index, edge_attr, W1, b1, W2, b2, Wmu, bmu, Wsig, bsig,
           Wmu2, bmu2, Wsig2, bsig2, edges, high):
    f32 = jnp.float32
    n_nodes = x.shape[0]
    src = edge_index[0]
    dst = edge_index[1]
    tail_ids = jnp.arange(n_nodes - 3, n_nodes, dtype=jnp.int32)
    slots = jnp.concatenate([edges[:, 0], edges[:, 1], tail_ids])
    slots_pad = jnp.concatenate(
        [slots, jnp.full((_NSLOT - 51,), -1, jnp.int32)])

    uniq = jnp.unique(slots, size=16, fill_value=-1)
    mask = jnp.any(src[:, None] == uniq[None, :], axis=1)
    n_rel = jnp.sum(mask.astype(jnp.int32))
    idx = jnp.nonzero(mask, size=_C, fill_value=0)[0]
    vm = (jnp.arange(_C) < n_rel).astype(f32)

    cs = src[idx]
    cxd = x[dst[idx]]
    cea = edge_attr[idx]
    xs = x[jnp.clip(slots_pad, 0, n_nodes - 1)]
    xu = x[jnp.clip(uniq, 0, n_nodes - 1)]

    w1a, w1b, w1c = W1[0:128], W1[128:256], W1[256:272]
    b1r = b1.reshape(1, 32)
    b2r = b2.reshape(1, 32)
    wh = jnp.zeros((320, 128), f32).at[:, 0].set(Wmu[:, 0]).at[:, 1].set(Wsig[:, 0])
    bhr = jnp.zeros((1, 128), f32).at[0, 0].set(bmu[0]).at[0, 1].set(bsig[0])
    wh2 = jnp.zeros((160, 128), f32).at[:, 0].set(Wmu2[:, 0]).at[:, 1].set(Wsig2[:, 0])
    bh2r = jnp.zeros((1, 128), f32).at[0, 0].set(bmu2[0]).at[0, 1].set(bsig2[0])

    csrow = cs.reshape(1, _C)
    cscol = cs.reshape(_C, 1)
    vmrow = vm.reshape(1, _C)
    vmcol = vm.reshape(_C, 1)
    urow = uniq.reshape(1, 16)
    ucol = uniq.reshape(16, 1)
    scol = slots_pad.reshape(_NSLOT, 1)

    def _fast(_):
        out = pl.pallas_call(
            _edge_head_body,
            out_shape=jax.ShapeDtypeStruct((32, 128), f32),
        )(csrow, cscol, vmrow, vmcol, urow, ucol, scol, xs, xu, cxd, cea,
          w1a, w1b, w1c, b1r, W2, b2r, wh, bhr, wh2, bh2r)
        r1 = out[0:24, 0]
        r2 = out[24:27, 0]
        return jnp.concatenate([r1 * high[:-3], r2 * high[-3:]])[None, :]

    def _dense(_):
        tmp = jnp.concatenate([x[src], x[dst], edge_attr], axis=1)
        msg = jax.nn.relu(tmp @ W1 + b1) @ W2 + b2
        conv = jax.ops.segment_sum(msg, src, num_segments=n_nodes)
        x_pp = jnp.concatenate([x, conv], axis=1).reshape(-1, n_nodes, 160)
        ef = jnp.concatenate([x_pp[:, edges[:, 0], :], x_pp[:, edges[:, 1], :]],
                             axis=2)
        alpha = jax.nn.softplus(ef @ Wmu + bmu + 1e-20)[..., 0] + 1e-20
        beta = jax.nn.softplus(ef @ Wsig + bsig + 1e-20)[..., 0] + 1e-20
        tail = x_pp[:, -3:, :]
        alpha2 = jax.nn.softplus(tail @ Wmu2 + bmu2 + 1e-20)[..., 0] + 1e-20
        beta2 = jax.nn.softplus(tail @ Wsig2 + bsig2 + 1e-20)[..., 0] + 1e-20
        dis_action = alpha / (alpha + beta) * high[:-3]
        order_act = alpha2 / (alpha2 + beta2) * high[-3:]
        return jnp.concatenate([dis_action, order_act], axis=-1)

    return lax.cond(n_rel <= _C, _fast, _dense, operand=None)


# speedup vs baseline: 10.8228x; 1.0245x over previous
import jax
import jax.numpy as jnp
from jax import lax
from jax.experimental import pallas as pl

_C = 1024
_NSLOT = 64


def _edge_head_body(csrow, cscol, vmrow, vmcol, urow, ucol, scol, xs, xu,
                    cxd, cea, w1a, w1b, w1c, b1r, w2, b2r, wh, bhr, wh2, bh2r,
                    out_ref):
    f32 = jnp.float32
    au = jnp.dot(xu[...], w1a[...], preferred_element_type=f32)
    mc = jnp.where(cscol[...] == urow[...], vmcol[...], 0.0)
    h = (jnp.dot(mc, au, preferred_element_type=f32)
         + jnp.dot(cxd[...], w1b[...], preferred_element_type=f32)
         + jnp.dot(cea[...], w1c[...], preferred_element_type=f32)
         + b1r[...])
    u = jnp.maximum(h, 0.0)
    mu = jnp.where(ucol[...] == csrow[...], vmrow[...], 0.0)
    accu = jnp.dot(mu, u, preferred_element_type=f32)
    cntu = jnp.dot(mu, jnp.ones((_C, 1), f32), preferred_element_type=f32)
    convu = jnp.dot(accu, w2[...], preferred_element_type=f32) + cntu * b2r[...]
    lk = jnp.where(scol[...] == urow[...], 1.0, 0.0)
    conv = jnp.dot(lk, convu, preferred_element_type=f32)
    xsa = xs[...]
    ef = jnp.concatenate([xsa[0:24], conv[0:24], xsa[24:48], conv[24:48]],
                         axis=1)
    z = jnp.dot(ef, wh[...], preferred_element_type=f32) + bhr[...]
    a = jax.nn.softplus(z + 1e-20) + 1e-20
    r1 = a[:, 0:1] / (a[:, 0:1] + a[:, 1:2])
    t = jnp.concatenate([xsa[48:56], conv[48:56]], axis=1)
    z2 = jnp.dot(t, wh2[...], preferred_element_type=f32) + bh2r[...]
    a2 = jax.nn.softplus(z2 + 1e-20) + 1e-20
    r2 = a2[:, 0:1] / (a2[:, 0:1] + a2[:, 1:2])
    out_ref[0:24, :] = jnp.broadcast_to(r1, (24, 128))
    out_ref[24:32, :] = jnp.broadcast_to(r2, (8, 128))


def kernel(x, edge_index, edge_attr, W1, b1, W2, b2, Wmu, bmu, Wsig, bsig,
           Wmu2, bmu2, Wsig2, bsig2, edges, high):
    f32 = jnp.float32
    n_nodes = x.shape[0]
    src = edge_index[0]
    dst = edge_index[1]
    tail_ids = jnp.arange(n_nodes - 3, n_nodes, dtype=jnp.int32)
    slots = jnp.concatenate([edges[:, 0], edges[:, 1], tail_ids])
    slots_pad = jnp.concatenate(
        [slots, jnp.full((_NSLOT - 51,), -1, jnp.int32)])

    uniq = jnp.unique(slots, size=16, fill_value=-1)
    mask = jnp.any(src[:, None] == uniq[None, :], axis=1)
    n_rel = jnp.sum(mask.astype(jnp.int32))
    idx = jnp.nonzero(mask, size=_C, fill_value=0)[0]
    vm = (jnp.arange(_C) < n_rel).astype(f32)

    cs = src[idx]
    cxd = x[dst[idx]]
    cea = edge_attr[idx]
    xs = x[jnp.clip(slots_pad, 0, n_nodes - 1)]
    xu = x[jnp.clip(uniq, 0, n_nodes - 1)]

    w1a, w1b, w1c = W1[0:128], W1[128:256], W1[256:272]
    b1r = b1.reshape(1, 32)
    b2r = b2.reshape(1, 32)
    wh = jnp.zeros((320, 128), f32).at[:, 0].set(Wmu[:, 0]).at[:, 1].set(Wsig[:, 0])
    bhr = jnp.zeros((1, 128), f32).at[0, 0].set(bmu[0]).at[0, 1].set(bsig[0])
    wh2 = jnp.zeros((160, 128), f32).at[:, 0].set(Wmu2[:, 0]).at[:, 1].set(Wsig2[:, 0])
    bh2r = jnp.zeros((1, 128), f32).at[0, 0].set(bmu2[0]).at[0, 1].set(bsig2[0])

    csrow = cs.reshape(1, _C)
    cscol = cs.reshape(_C, 1)
    vmrow = vm.reshape(1, _C)
    vmcol = vm.reshape(_C, 1)
    urow = uniq.reshape(1, 16)
    ucol = uniq.reshape(16, 1)
    scol = slots_pad.reshape(_NSLOT, 1)

    def _fast(_):
        out = pl.pallas_call(
            _edge_head_body,
            out_shape=jax.ShapeDtypeStruct((32, 128), f32),
        )(csrow, cscol, vmrow, vmcol, urow, ucol, scol, xs, xu, cxd, cea,
          w1a, w1b, w1c, b1r, W2, b2r, wh, bhr, wh2, bh2r)
        r1 = out[0:24, 0]
        r2 = out[24:27, 0]
        return jnp.concatenate([r1 * high[:-3], r2 * high[-3:]])[None, :]

    def _dense(_):
        tmp = jnp.concatenate([x[src], x[dst], edge_attr], axis=1)
        msg = jax.nn.relu(tmp @ W1 + b1) @ W2 + b2
        conv = jax.ops.segment_sum(msg, src, num_segments=n_nodes)
        x_pp = jnp.concatenate([x, conv], axis=1).reshape(-1, n_nodes, 160)
        ef = jnp.concatenate([x_pp[:, edges[:, 0], :], x_pp[:, edges[:, 1], :]],
                             axis=2)
        alpha = jax.nn.softplus(ef @ Wmu + bmu + 1e-20)[..., 0] + 1e-20
        beta = jax.nn.softplus(ef @ Wsig + bsig + 1e-20)[..., 0] + 1e-20
        tail = x_pp[:, -3:, :]
        alpha2 = jax.nn.softplus(tail @ Wmu2 + bmu2 + 1e-20)[..., 0] + 1e-20
        beta2 = jax.nn.softplus(tail @ Wsig2 + bsig2 + 1e-20)[..., 0] + 1e-20
        dis_action = alpha / (alpha + beta) * high[:-3]
        order_act = alpha2 / (alpha2 + beta2) * high[-3:]
        return jnp.concatenate([dis_action, order_act], axis=-1)

    return lax.cond(n_rel <= _C, _fast, _dense, operand=None)
